# baseline (speedup 1.0000x reference)
import functools

import jax
import jax.numpy as jnp
from jax import lax
from jax.experimental import pallas as pl
from jax.experimental.pallas import tpu as pltpu
from jax.experimental.pallas import tpu_sc as plsc

B, N, D = 4, 2048, 64
UNITS, LAT, K, H = 64, 32, 16, 4
CH = UNITS * H

RBLK = 256
NBLK = N // RBLK

SC_CORES = 2
SC_SUBCORES = 16
SC_WORKERS = SC_CORES * SC_SUBCORES
TOTAL_IDX = B * N * K
PER_W = TOTAL_IDX // SC_WORKERS
CHUNK = 128
TW = 64


CCH = 256
NCH = N // CCH


def _topk_body(x_blk_ref, x_full_ref, idx_ref, key_scr):
    b = pl.program_id(0)
    xb = x_blk_ref[0]
    ones_col = jnp.ones((D, 1), jnp.float32)
    for c in range(NCH):
        xf_c = x_full_ref[0, c * CCH:(c + 1) * CCH, :]
        sq_c = lax.dot_general(
            xf_c * xf_c, ones_col, (((1,), (0,)), ((), ())),
            preferred_element_type=jnp.float32,
            precision=lax.Precision.HIGHEST)
        innerT = lax.dot_general(
            xf_c, xb, (((1,), (1,)), ((), ())),
            preferred_element_type=jnp.float32)
        key_scr[c * CCH:(c + 1) * CCH, :] = sq_c - 2.0 * innerT

    kiota = lax.broadcasted_iota(jnp.int32, (K, RBLK), 0)

    def step(t, carry):
        m_prev, i_prev, acc = carry
        m = i_star = None
        for c in range(NCH):
            iota_c = lax.broadcasted_iota(jnp.int32, (CCH, RBLK), 0) + c * CCH
            chunk = key_scr[c * CCH:(c + 1) * CCH, :]
            valid = (chunk > m_prev) | ((chunk == m_prev) & (iota_c > i_prev))
            vv = jnp.where(valid, chunk, jnp.inf)
            mc = jnp.min(vv, axis=0, keepdims=True)
            ic = jnp.min(jnp.where(vv == mc, iota_c, N), axis=0, keepdims=True)
            if c == 0:
                m, i_star = mc, ic
            else:
                better = (mc < m) | ((mc == m) & (ic < i_star))
                m = jnp.where(better, mc, m)
                i_star = jnp.where(better, ic, i_star)
        acc = jnp.where(kiota == t, i_star, acc)
        return m, i_star, acc

    m0 = jnp.full((1, RBLK), -jnp.inf, jnp.float32)
    i0 = jnp.full((1, RBLK), -1, jnp.int32)
    _, _, acc = lax.fori_loop(
        0, K, step, (m0, i0, jnp.zeros((K, RBLK), jnp.int32)))
    idx_ref[0, 0] = acc + b * N


NCHK = PER_W // CHUNK
NBUF = 4


def _sc_gather(table, idx2d):
    mesh = plsc.VectorSubcoreMesh(core_axis_name="c", subcore_axis_name="s")

    @functools.partial(
        pl.kernel,
        out_type=jax.ShapeDtypeStruct((TOTAL_IDX, TW), jnp.float32),
        mesh=mesh,
        compiler_params=pltpu.CompilerParams(use_tc_tiling_on_sc=False),
        scratch_types=(
            [pltpu.VMEM((NCHK, CHUNK), jnp.int32)]
            + [pltpu.VMEM((CHUNK, TW), jnp.float32)] * NBUF
            + [pltpu.SemaphoreType.DMA] * (2 * NBUF)
        ),
    )
    def gather(table_hbm, idx_hbm, out_hbm, idx_v, *rest):
        bufs = rest[:NBUF]
        gsem = rest[NBUF:2 * NBUF]
        wsem = rest[2 * NBUF:]
        wid = lax.axis_index("s") * SC_CORES + lax.axis_index("c")
        row0 = wid * NCHK
        pltpu.sync_copy(idx_hbm.at[pl.ds(row0, NCHK)], idx_v)

        gd = [None] * NCHK
        wd = [None] * NCHK
        for i in range(NCHK):
            b = i % NBUF
            if i >= NBUF:
                wd[i - NBUF].wait()
            gd[i] = pltpu.async_copy(table_hbm.at[idx_v.at[i]], bufs[b], gsem[b])
            if i >= 1:
                gd[i - 1].wait()
                wd[i - 1] = pltpu.async_copy(
                    bufs[(i - 1) % NBUF],
                    out_hbm.at[pl.ds((row0 + i - 1) * CHUNK, CHUNK)],
                    wsem[(i - 1) % NBUF])
        gd[NCHK - 1].wait()
        wd[NCHK - 1] = pltpu.async_copy(
            bufs[(NCHK - 1) % NBUF],
            out_hbm.at[pl.ds((row0 + NCHK - 1) * CHUNK, CHUNK)],
            wsem[(NCHK - 1) % NBUF])
        for i in range(NCHK - NBUF, NCHK):
            wd[i].wait()

    return gather(table, idx2d)


def _attn_body(x_blk_ref, xnb_ref, mq_ref, mk_ref, mv_ref,
               wp0_ref, wp1_ref, wa0c_ref, wa1_ref, wout_ref, out_ref):
    xb = x_blk_ref[0]
    xnb = xnb_ref[0][:, :D]
    mm = functools.partial(jnp.dot, preferred_element_type=jnp.float32)

    edges = (xnb.reshape(RBLK, K, D) - xb[:, None, :]).reshape(RBLK * K, D)
    h1 = jnp.maximum(mm(edges, wp0_ref[...]), 0.0)
    pos = mm(h1, wp1_ref[...])
    posa = mm(pos, wa0c_ref[...])

    qa = mm(xb, mq_ref[...])
    ka = mm(xnb, mk_ref[...])
    vnb = mm(xnb, mv_ref[...])

    preact = (qa[:, None, :] - ka.reshape(RBLK, K, H * LAT)).reshape(
        RBLK * K, H * LAT) + posa
    act = jnp.maximum(preact, 0.0)
    logits = jnp.concatenate(
        [mm(act[:, h * LAT:(h + 1) * LAT], wa1_ref[h]) for h in range(H)],
        axis=1)

    a = logits.reshape(RBLK, K, CH)
    amax = jnp.max(a, axis=1, keepdims=True)
    e = jnp.exp(a - amax)
    att = e / jnp.sum(e, axis=1, keepdims=True)

    pos_t = jnp.concatenate([pos] * H, axis=1)
    val = (vnb + pos_t).reshape(RBLK, K, CH)
    mh = jnp.sum(att * val, axis=1)
    out_ref[0] = mm(mh, wout_ref[...])


def kernel(inputs, W_p0, W_p1, W_in, W_q, W_k, W_v, W_a0, W_a1, W_out):
    x = inputs

    f32 = jnp.float32
    W_inq = W_in @ W_q
    W_ink = W_in @ W_k
    Mq = jnp.einsum("dhx,hxl->dhl", W_inq.reshape(D, H, UNITS), W_a0).reshape(D, H * LAT)
    Mk = jnp.einsum("dhx,hxl->dhl", W_ink.reshape(D, H, UNITS), W_a0).reshape(D, H * LAT)
    Mv = W_in @ W_v
    Wa0c = jnp.transpose(W_a0, (1, 0, 2)).reshape(UNITS, H * LAT)

    idx_t = pl.pallas_call(
        _topk_body,
        grid=(B, NBLK),
        in_specs=[
            pl.BlockSpec((1, RBLK, D), lambda b, r: (b, r, 0)),
            pl.BlockSpec((1, N, D), lambda b, r: (b, 0, 0)),
        ],
        out_specs=pl.BlockSpec((1, 1, K, RBLK), lambda b, r: (b, r, 0, 0)),
        out_shape=jax.ShapeDtypeStruct((B, NBLK, K, RBLK), jnp.int32),
        scratch_shapes=[pltpu.VMEM((N, RBLK), jnp.float32)],
    )(x, x)
    idx = jnp.transpose(idx_t, (0, 1, 3, 2)).reshape(B, N, K)
    table = jnp.concatenate(
        [x.reshape(B * N, D), jnp.zeros((B * N, max(TW - D, 0)), f32)], axis=1) if TW > D else x.reshape(B * N, D)
    xnb = _sc_gather(table, idx.reshape(TOTAL_IDX // CHUNK, CHUNK))

    wspec = lambda shape: pl.BlockSpec(shape, lambda b, r: tuple(0 for _ in shape))
    out = pl.pallas_call(
        _attn_body,
        grid=(B, NBLK),
        in_specs=[
            pl.BlockSpec((1, RBLK, D), lambda b, r: (b, r, 0)),
            pl.BlockSpec((1, RBLK * K, TW), lambda b, r: (b, r, 0)),
            wspec((D, H * LAT)),
            wspec((D, H * LAT)),
            wspec((D, CH)),
            wspec((D, LAT)),
            wspec((LAT, UNITS)),
            wspec((UNITS, H * LAT)),
            wspec((H, LAT, UNITS)),
            wspec((CH, UNITS)),
        ],
        out_specs=pl.BlockSpec((1, RBLK, UNITS), lambda b, r: (b, r, 0)),
        out_shape=jax.ShapeDtypeStruct((B, N, UNITS), f32),
    )(x, xnb.reshape(B, N * K, TW), Mq, Mk, Mv,
      W_p0, W_p1, Wa0c, W_a1, W_out)
    return out


# speedup vs baseline: 1.6199x; 1.6199x over previous
import functools

import jax
import jax.numpy as jnp
from jax import lax
from jax.experimental import pallas as pl
from jax.experimental.pallas import tpu as pltpu
from jax.experimental.pallas import tpu_sc as plsc

B, N, D = 4, 2048, 64
UNITS, LAT, K, H = 64, 32, 16, 4
CH = UNITS * H

RBLK = 256
NBLK = N // RBLK

SC_CORES = 2
SC_SUBCORES = 16
SC_WORKERS = SC_CORES * SC_SUBCORES
TOTAL_IDX = B * N * K
PER_W = TOTAL_IDX // SC_WORKERS
CHUNK = 128
TW = 128


CCH = 256
NCH = N // CCH


def _topk_body(x_blk_ref, x_full_ref, idx_ref, key_scr):
    b = pl.program_id(0)
    xb = x_blk_ref[0]
    ones_col = jnp.ones((D, 1), jnp.float32)
    for c in range(NCH):
        xf_c = x_full_ref[0, c * CCH:(c + 1) * CCH, :]
        sq_c = lax.dot_general(
            xf_c * xf_c, ones_col, (((1,), (0,)), ((), ())),
            preferred_element_type=jnp.float32,
            precision=lax.Precision.HIGHEST)
        innerT = lax.dot_general(
            xf_c, xb, (((1,), (1,)), ((), ())),
            preferred_element_type=jnp.float32)
        key_scr[c * CCH:(c + 1) * CCH, :] = sq_c - 2.0 * innerT

    kiota = lax.broadcasted_iota(jnp.int32, (K, RBLK), 0)

    def step(t, carry):
        i_prev, acc = carry
        fv = fi = None
        for c in range(NCH):
            iota_c = lax.broadcasted_iota(jnp.int32, (CCH, RBLK), 0) + c * CCH
            chunk = key_scr[c * CCH:(c + 1) * CCH, :]
            masked = jnp.where(iota_c == i_prev, jnp.inf, chunk)
            key_scr[c * CCH:(c + 1) * CCH, :] = masked
            if c == 0:
                fv, fi = masked, iota_c
            else:
                better = masked < fv
                fv = jnp.where(better, masked, fv)
                fi = jnp.where(better, iota_c, fi)
        rows = CCH
        while rows > 8:
            h = rows // 2
            av, ai = fv[:h], fi[:h]
            bv, bi = fv[h:], fi[h:]
            better = (bv < av) | ((bv == av) & (bi < ai))
            fv = jnp.where(better, bv, av)
            fi = jnp.where(better, bi, ai)
            rows = h
        mn = jnp.min(fv, axis=0, keepdims=True)
        i_star = jnp.min(jnp.where(fv == mn, fi, N * B), axis=0, keepdims=True)
        acc = jnp.where(kiota == t, i_star, acc)
        return i_star, acc

    i0 = jnp.full((1, RBLK), -1, jnp.int32)
    _, acc = lax.fori_loop(
        0, K, step, (i0, jnp.zeros((K, RBLK), jnp.int32)))
    idx_ref[0, 0] = acc + b * N


NCHK = PER_W // CHUNK
NBUF = 4


def _sc_gather(table, idx2d):
    mesh = plsc.VectorSubcoreMesh(core_axis_name="c", subcore_axis_name="s")
    nrows = idx2d.shape[0]
    nchk = nrows // SC_WORKERS

    @functools.partial(
        pl.kernel,
        out_type=jax.ShapeDtypeStruct((nrows * CHUNK, TW), jnp.float32),
        mesh=mesh,
        scratch_types=(
            [pltpu.VMEM((nchk, CHUNK), jnp.int32)]
            + [pltpu.VMEM((CHUNK, TW), jnp.float32)] * NBUF
            + [pltpu.SemaphoreType.DMA] * (2 * NBUF)
        ),
    )
    def gather(table_hbm, idx_hbm, out_hbm, idx_v, *rest):
        NCHK = nchk
        bufs = rest[:NBUF]
        gsem = rest[NBUF:2 * NBUF]
        wsem = rest[2 * NBUF:]
        wid = lax.axis_index("s") * SC_CORES + lax.axis_index("c")
        row0 = wid * NCHK
        pltpu.sync_copy(idx_hbm.at[pl.ds(row0, NCHK)], idx_v)

        gd = [None] * NCHK
        wd = [None] * NCHK
        for i in range(NCHK):
            b = i % NBUF
            if i >= NBUF:
                wd[i - NBUF].wait()
            gd[i] = pltpu.async_copy(table_hbm.at[idx_v.at[i]], bufs[b], gsem[b])
            if i >= 1:
                gd[i - 1].wait()
                wd[i - 1] = pltpu.async_copy(
                    bufs[(i - 1) % NBUF],
                    out_hbm.at[pl.ds((row0 + i - 1) * CHUNK, CHUNK)],
                    wsem[(i - 1) % NBUF])
        gd[NCHK - 1].wait()
        wd[NCHK - 1] = pltpu.async_copy(
            bufs[(NCHK - 1) % NBUF],
            out_hbm.at[pl.ds((row0 + NCHK - 1) * CHUNK, CHUNK)],
            wsem[(NCHK - 1) % NBUF])
        for i in range(NCHK - NBUF, NCHK):
            wd[i].wait()

    return gather(table, idx2d)


def _attn_body(x_blk_ref, xnb_ref, mq_ref, mk_ref, mv_ref,
               wp0_ref, wp1_ref, wa0c_ref, wa1_ref, wout_ref, out_ref):
    xb = x_blk_ref[0]
    xnb = xnb_ref[0][:, :D]
    mm = functools.partial(jnp.dot, preferred_element_type=jnp.float32)

    edges = (xnb.reshape(RBLK, K, D) - xb[:, None, :]).reshape(RBLK * K, D)
    h1 = jnp.maximum(mm(edges, wp0_ref[...]), 0.0)
    pos = mm(h1, wp1_ref[...])
    posa = mm(pos, wa0c_ref[...])

    qa = mm(xb, mq_ref[...])
    ka = mm(xnb, mk_ref[...])
    vnb = mm(xnb, mv_ref[...])

    preact = (qa[:, None, :] - ka.reshape(RBLK, K, H * LAT)).reshape(
        RBLK * K, H * LAT) + posa
    act = jnp.maximum(preact, 0.0)
    logits = jnp.concatenate(
        [mm(act[:, h * LAT:(h + 1) * LAT], wa1_ref[h]) for h in range(H)],
        axis=1)

    a = logits.reshape(RBLK, K, CH)
    amax = jnp.max(a, axis=1, keepdims=True)
    e = jnp.exp(a - amax)
    att = e / jnp.sum(e, axis=1, keepdims=True)

    pos_t = jnp.concatenate([pos] * H, axis=1)
    val = (vnb + pos_t).reshape(RBLK, K, CH)
    mh = jnp.sum(att * val, axis=1)
    out_ref[0] = mm(mh, wout_ref[...])


def kernel(inputs, W_p0, W_p1, W_in, W_q, W_k, W_v, W_a0, W_a1, W_out):
    x = inputs

    f32 = jnp.float32
    W_inq = W_in @ W_q
    W_ink = W_in @ W_k
    Mq = jnp.einsum("dhx,hxl->dhl", W_inq.reshape(D, H, UNITS), W_a0).reshape(D, H * LAT)
    Mk = jnp.einsum("dhx,hxl->dhl", W_ink.reshape(D, H, UNITS), W_a0).reshape(D, H * LAT)
    Mv = W_in @ W_v
    Wa0c = jnp.transpose(W_a0, (1, 0, 2)).reshape(UNITS, H * LAT)

    wspec = lambda shape: pl.BlockSpec(shape, lambda b, r: tuple(0 for _ in shape))
    HB = 2
    outs = []
    for hh in range(B // HB):
        xh = x[hh * HB:(hh + 1) * HB]
        idx_t = pl.pallas_call(
            _topk_body,
            grid=(HB, NBLK),
            in_specs=[
                pl.BlockSpec((1, RBLK, D), lambda b, r: (b, r, 0)),
                pl.BlockSpec((1, N, D), lambda b, r: (b, 0, 0)),
            ],
            out_specs=pl.BlockSpec((1, 1, K, RBLK), lambda b, r: (b, r, 0, 0)),
            out_shape=jax.ShapeDtypeStruct((HB, NBLK, K, RBLK), jnp.int32),
            scratch_shapes=[pltpu.VMEM((N, RBLK), jnp.float32)],
        )(xh, xh)
        idx = jnp.transpose(idx_t, (0, 1, 3, 2)).reshape(HB, N, K)
        table = jnp.concatenate(
            [xh.reshape(HB * N, D), jnp.zeros((HB * N, TW - D), f32)], axis=1)
        xnb = _sc_gather(table, idx.reshape(HB * N * K // CHUNK, CHUNK))

        out_h = pl.pallas_call(
            _attn_body,
            grid=(HB, NBLK),
            in_specs=[
                pl.BlockSpec((1, RBLK, D), lambda b, r: (b, r, 0)),
                pl.BlockSpec((1, RBLK * K, TW), lambda b, r: (b, r, 0)),
                wspec((D, H * LAT)),
                wspec((D, H * LAT)),
                wspec((D, CH)),
                wspec((D, LAT)),
                wspec((LAT, UNITS)),
                wspec((UNITS, H * LAT)),
                wspec((H, LAT, UNITS)),
                wspec((CH, UNITS)),
            ],
            out_specs=pl.BlockSpec((1, RBLK, UNITS), lambda b, r: (b, r, 0)),
            out_shape=jax.ShapeDtypeStruct((HB, N, UNITS), f32),
        )(xh, xnb.reshape(HB, N * K, TW), Mq, Mk, Mv,
          W_p0, W_p1, Wa0c, W_a1, W_out)
        outs.append(out_h)
    return jnp.concatenate(outs, axis=0)
